# baseline (speedup 1.0000x reference)
import functools

import jax
import jax.numpy as jnp
from jax import lax
from jax.experimental import pallas as pl
from jax.experimental.pallas import tpu as pltpu
from jax.experimental.pallas import tpu_sc as plsc

N = 10000
DIM = 128
E = 320000

NC = 2
NS = 16
NW = NC * NS

ACC_W = 144
N_PAD = 10000
ROWS_PER_SUB = N_PAD // NS
CHUNK = 40
EDGES_PER_W = E // NW
N_PAIRS = EDGES_PER_W // (2 * CHUNK)

ROW_BLK = 2000


def _node_mm_body(x_ref, wh_ref, bh_ref, a_ref, b_ref, c_ref):
    y = jnp.dot(x_ref[...], wh_ref[...], preferred_element_type=jnp.float32)
    y = y + bh_ref[...]
    a_ref[...] = y[:, :DIM].astype(jnp.bfloat16)
    b_ref[...] = y[:, DIM:2 * DIM].astype(jnp.bfloat16)
    c_ref[...] = y[:, 2 * DIM:]


_node_mm = pl.pallas_call(
    _node_mm_body,
    grid=(N // ROW_BLK,),
    in_specs=[
        pl.BlockSpec((ROW_BLK, DIM), lambda i: (i, 0)),
        pl.BlockSpec((DIM, 3 * DIM), lambda i: (0, 0)),
        pl.BlockSpec((1, 3 * DIM), lambda i: (0, 0)),
    ],
    out_specs=[
        pl.BlockSpec((ROW_BLK, DIM), lambda i: (i, 0)),
        pl.BlockSpec((ROW_BLK, DIM), lambda i: (i, 0)),
        pl.BlockSpec((ROW_BLK, DIM), lambda i: (i, 0)),
    ],
    out_shape=[
        jax.ShapeDtypeStruct((N, DIM), jnp.bfloat16),
        jax.ShapeDtypeStruct((N, DIM), jnp.bfloat16),
        jax.ShapeDtypeStruct((N, DIM), jnp.float32),
    ],
)


_sc_mesh = plsc.VectorSubcoreMesh(core_axis_name="c", subcore_axis_name="s")


@functools.partial(
    pl.kernel,
    out_type=jax.ShapeDtypeStruct((NC, N_PAD, ACC_W), jnp.float32),
    mesh=_sc_mesh,
    compiler_params=pltpu.CompilerParams(needs_layout_passes=False,
                                         use_tc_tiling_on_sc=False),
    scratch_types=[
        pltpu.VMEM((CHUNK,), jnp.int32),
        pltpu.VMEM((CHUNK,), jnp.int32),
        pltpu.VMEM((CHUNK,), jnp.int32),
        pltpu.VMEM((CHUNK,), jnp.int32),
        pltpu.VMEM((CHUNK, DIM), jnp.bfloat16),
        pltpu.VMEM((CHUNK, DIM), jnp.bfloat16),
        pltpu.VMEM((CHUNK, DIM), jnp.float32),
        pltpu.VMEM((CHUNK, DIM), jnp.bfloat16),
        pltpu.VMEM((CHUNK, DIM), jnp.bfloat16),
        pltpu.VMEM((CHUNK, DIM), jnp.float32),
        pltpu.VMEM((CHUNK, ACC_W), jnp.float32),
        pltpu.VMEM((2 * DIM,), jnp.bfloat16),
        pltpu.VMEM((16,), jnp.float32),
        pltpu.VMEM_SHARED((N_PAD, ACC_W), jnp.float32),
        pltpu.SemaphoreType.DMA,
        pltpu.SemaphoreType.DMA,
        pltpu.SemaphoreType.DMA,
        pltpu.SemaphoreType.DMA,
        pltpu.SemaphoreType.DMA,
        pltpu.SemaphoreType.DMA,
    ],
)
def _edge_sc(row_hbm, col_hbm, a_hbm, b_hbm, c_hbm, w2_hbm, b2_hbm, out_hbm,
             row_v0, col_v0, row_v1, col_v1,
             a_v0, b_v0, c_v0, a_v1, b_v1, c_v1,
             m_v, w2_v, b2_v, acc,
             sem_a0, sem_b0, sem_c0, sem_a1, sem_b1, sem_c1):
    c = lax.axis_index("c")
    s = lax.axis_index("s")
    wid = s * NC + c

    zeros16 = jnp.zeros((16,), jnp.float32)

    def _zrow(e, carry):
        for k in range(ACC_W // 16):
            m_v[e, pl.ds(k * 16, 16)] = zeros16
        return carry

    lax.fori_loop(0, CHUNK, _zrow, 0)

    def _zacc(j, carry):
        pltpu.sync_copy(m_v.at[pl.ds(0, 25)],
                        acc.at[pl.ds(s * ROWS_PER_SUB + j * 25, 25)])
        return carry

    lax.fori_loop(0, ROWS_PER_SUB // 25, _zacc, 0)

    deg16 = jnp.where(lax.iota(jnp.int32, 16) == 0, 1.0, 0.0).astype(jnp.float32)

    def _deg1(e, carry):
        m_v[e, pl.ds(DIM, 16)] = deg16
        return carry

    lax.fori_loop(0, CHUNK, _deg1, 0)

    pltpu.sync_copy(w2_hbm, w2_v)
    pltpu.sync_copy(b2_hbm, b2_v)
    w2p = [w2_v[pl.ds(k * 32, 32)] for k in range(DIM // 32)]
    b2s = b2_v[...][0]

    plsc.subcore_barrier()

    ebase = wid * EDGES_PER_W

    def _fetch(base, row_v, col_v, a_v, b_v, c_v, sem_a, sem_b, sem_c):
        pltpu.sync_copy(row_hbm.at[pl.ds(base, CHUNK)], row_v)
        pltpu.sync_copy(col_hbm.at[pl.ds(base, CHUNK)], col_v)
        pltpu.async_copy(a_hbm.at[row_v], a_v, sem_a)
        pltpu.async_copy(b_hbm.at[col_v], b_v, sem_b)
        pltpu.async_copy(c_hbm.at[col_v], c_v, sem_c)

    def _drain(a_v, b_v, c_v, sem_a, sem_b, sem_c):
        pltpu.make_async_copy(a_hbm.at[pl.ds(0, CHUNK)], a_v, sem_a).wait()
        pltpu.make_async_copy(b_hbm.at[pl.ds(0, CHUNK)], b_v, sem_b).wait()
        pltpu.make_async_copy(c_hbm.at[pl.ds(0, CHUNK)], c_v, sem_c).wait()

    def _compute(row_v, a_v, b_v, c_v):
        @plsc.parallel_loop(0, CHUNK, 1, unroll=4)
        def _edge(e):
            parts = []
            for k in range(DIM // 32):
                a32 = a_v[e, pl.ds(k * 32, 32)]
                b32 = b_v[e, pl.ds(k * 32, 32)]
                t = jnp.maximum(a32 + b32, 0) * w2p[k]
                lo, hi = plsc.unpack(t, format=plsc.PackFormat.INTERLEAVED)
                parts.append(lo + hi)
            dot = jnp.sum((parts[0] + parts[1]) + (parts[2] + parts[3])) + b2s
            zv = jnp.broadcast_to(dot, (16,))
            gv = 1.0 / (1.0 + jnp.exp(-zv))
            for k in range(DIM // 16):
                cv = c_v[e, pl.ds(k * 16, 16)]
                m_v[e, pl.ds(k * 16, 16)] = gv * cv

        pltpu.sync_copy(m_v, acc.at[row_v], add=True)

    _fetch(ebase, row_v0, col_v0, a_v0, b_v0, c_v0, sem_a0, sem_b0, sem_c0)

    def _pair(i, carry):
        base1 = ebase + (2 * i + 1) * CHUNK
        _fetch(base1, row_v1, col_v1, a_v1, b_v1, c_v1, sem_a1, sem_b1, sem_c1)
        _drain(a_v0, b_v0, c_v0, sem_a0, sem_b0, sem_c0)
        _compute(row_v0, a_v0, b_v0, c_v0)
        base2 = jnp.minimum(ebase + (2 * i + 2) * CHUNK, E - CHUNK)
        _fetch(base2, row_v0, col_v0, a_v0, b_v0, c_v0, sem_a0, sem_b0, sem_c0)
        _drain(a_v1, b_v1, c_v1, sem_a1, sem_b1, sem_c1)
        _compute(row_v1, a_v1, b_v1, c_v1)
        return carry

    lax.fori_loop(0, N_PAIRS, _pair, 0)
    _drain(a_v0, b_v0, c_v0, sem_a0, sem_b0, sem_c0)

    plsc.subcore_barrier()

    def _cp(j, carry):
        r0 = s * ROWS_PER_SUB + j * 25
        pltpu.sync_copy(acc.at[pl.ds(r0, 25)], m_v.at[pl.ds(0, 25)])
        pltpu.sync_copy(m_v.at[pl.ds(0, 25)], out_hbm.at[c, pl.ds(r0, 25)])
        return carry

    lax.fori_loop(0, ROWS_PER_SUB // 25, _cp, 0)


def _final_body(x_ref, acc_ref, gamma_ref, beta_ref, o_ref):
    sacc = acc_ref[0] + acc_ref[1]
    msg = sacc[:, :DIM]
    deg = jnp.maximum(sacc[:, DIM:DIM + 1], 1.0)
    y = x_ref[...] + msg / deg
    mean = jnp.mean(y, axis=1, keepdims=True)
    cent = y - mean
    var = jnp.mean(cent * cent, axis=1, keepdims=True)
    yn = cent * lax.rsqrt(var + 1e-5)
    o_ref[...] = jnp.maximum(gamma_ref[...] * yn + beta_ref[...], 0.0)


_final = pl.pallas_call(
    _final_body,
    grid=(N // ROW_BLK,),
    in_specs=[
        pl.BlockSpec((ROW_BLK, DIM), lambda i: (i, 0)),
        pl.BlockSpec((NC, ROW_BLK, ACC_W), lambda i: (0, i, 0)),
        pl.BlockSpec((1, DIM), lambda i: (0, 0)),
        pl.BlockSpec((1, DIM), lambda i: (0, 0)),
    ],
    out_specs=pl.BlockSpec((ROW_BLK, DIM), lambda i: (i, 0)),
    out_shape=jax.ShapeDtypeStruct((N, DIM), jnp.float32),
)


def kernel(x, edge_index, W, W1, b1, W2, b2, gamma, beta):
    wh = jnp.concatenate([W1[:DIM], W1[DIM:], W], axis=1)
    bh = jnp.concatenate([jnp.zeros((DIM,), jnp.float32), b1,
                          jnp.zeros((DIM,), jnp.float32)])[None]
    w2bf = jnp.concatenate([W2[:, 0],
                            jnp.zeros((DIM,), jnp.float32)]).astype(jnp.bfloat16)
    b2p = jnp.broadcast_to(b2, (16,)).astype(jnp.float32)

    a_arr, b_arr, c_arr = _node_mm(x, wh, bh)
    acc = _edge_sc(edge_index[0], edge_index[1], a_arr, b_arr, c_arr,
                   w2bf, b2p)
    return _final(x, acc, gamma[None], beta[None])


# speedup vs baseline: 1.2813x; 1.2813x over previous
import functools

import jax
import jax.numpy as jnp
from jax import lax
from jax.experimental import pallas as pl
from jax.experimental.pallas import tpu as pltpu
from jax.experimental.pallas import tpu_sc as plsc

N = 10000
DIM = 128
E = 320000

NC = 2
NS = 16
NW = NC * NS

ACC_W = 144
N_PAD = 10000
ROWS_PER_SUB = N_PAD // NS
CHUNK = 40
EDGES_PER_W = E // NW
N_PAIRS = EDGES_PER_W // (2 * CHUNK)

ROW_BLK = 2000


def _node_mm_body(x_ref, wh_ref, bh_ref, a_ref, bc_ref):
    y = jnp.dot(x_ref[...], wh_ref[...], preferred_element_type=jnp.float32)
    y = y + bh_ref[...]
    a_ref[...] = y[:, :DIM]
    bc_ref[...] = y[:, DIM:]


_node_mm = pl.pallas_call(
    _node_mm_body,
    grid=(N // ROW_BLK,),
    in_specs=[
        pl.BlockSpec((ROW_BLK, DIM), lambda i: (i, 0)),
        pl.BlockSpec((DIM, 3 * DIM), lambda i: (0, 0)),
        pl.BlockSpec((1, 3 * DIM), lambda i: (0, 0)),
    ],
    out_specs=[
        pl.BlockSpec((ROW_BLK, DIM), lambda i: (i, 0)),
        pl.BlockSpec((ROW_BLK, 2 * DIM), lambda i: (i, 0)),
    ],
    out_shape=[
        jax.ShapeDtypeStruct((N, DIM), jnp.float32),
        jax.ShapeDtypeStruct((N, 2 * DIM), jnp.float32),
    ],
)


_sc_mesh = plsc.VectorSubcoreMesh(core_axis_name="c", subcore_axis_name="s")


@functools.partial(
    pl.kernel,
    out_type=jax.ShapeDtypeStruct((NC, N_PAD, ACC_W), jnp.float32),
    mesh=_sc_mesh,
    compiler_params=pltpu.CompilerParams(needs_layout_passes=False,
                                         use_tc_tiling_on_sc=False),
    scratch_types=[
        pltpu.VMEM((2, CHUNK), jnp.int32),
        pltpu.VMEM((2, CHUNK), jnp.int32),
        pltpu.VMEM((CHUNK, DIM), jnp.float32),
        pltpu.VMEM((CHUNK, 2 * DIM), jnp.float32),
        pltpu.VMEM((CHUNK, DIM), jnp.float32),
        pltpu.VMEM((CHUNK, 2 * DIM), jnp.float32),
        pltpu.VMEM((CHUNK, ACC_W), jnp.float32),
        pltpu.VMEM((ACC_W,), jnp.float32),
        pltpu.VMEM_SHARED((N_PAD, ACC_W), jnp.float32),
        pltpu.SemaphoreType.DMA,
        pltpu.SemaphoreType.DMA,
        pltpu.SemaphoreType.DMA,
        pltpu.SemaphoreType.DMA,
        pltpu.SemaphoreType.DMA,
        pltpu.SemaphoreType.DMA,
    ],
)
def _edge_sc(idx_hbm, a_hbm, bc_hbm, w2b2_hbm, out_hbm,
             idx_v0, idx_v1, a_v0, bc_v0, a_v1, bc_v1,
             m_v, w2_v, acc, sem_a0, sem_b0, sem_a1, sem_b1, sem_i0, sem_i1):
    c = lax.axis_index("c")
    s = lax.axis_index("s")
    wid = s * NC + c

    zeros16 = jnp.zeros((16,), jnp.float32)

    def _zrow(e, carry):
        for k in range(ACC_W // 16):
            m_v[e, pl.ds(k * 16, 16)] = zeros16
        return carry

    lax.fori_loop(0, CHUNK, _zrow, 0)

    def _zacc(j, carry):
        pltpu.sync_copy(m_v.at[pl.ds(0, 25)],
                        acc.at[pl.ds(s * ROWS_PER_SUB + j * 25, 25)])
        return carry

    lax.fori_loop(0, ROWS_PER_SUB // 25, _zacc, 0)

    deg16 = jnp.where(lax.iota(jnp.int32, 16) == 0, 1.0, 0.0).astype(jnp.float32)

    def _deg1(e, carry):
        m_v[e, pl.ds(DIM, 16)] = deg16
        return carry

    lax.fori_loop(0, CHUNK, _deg1, 0)

    pltpu.sync_copy(w2b2_hbm, w2_v)
    w2r = [w2_v[pl.ds(k * 16, 16)] for k in range(DIM // 16)]
    b2s = w2_v[pl.ds(DIM, 16)][0]

    plsc.subcore_barrier()

    cbase = wid * (EDGES_PER_W // CHUNK)
    cmax = cbase + EDGES_PER_W // CHUNK - 1

    def _fetch_idx(cnum, idx_v, sem):
        pltpu.async_copy(idx_hbm.at[cnum], idx_v, sem)

    def _wait_idx(idx_v, sem):
        pltpu.make_async_copy(idx_hbm.at[0], idx_v, sem).wait()

    def _issue(idx_v, a_v, bc_v, sem_a, sem_b):
        pltpu.async_copy(a_hbm.at[idx_v.at[0]], a_v, sem_a)
        pltpu.async_copy(bc_hbm.at[idx_v.at[1]], bc_v, sem_b)

    def _drain(a_v, bc_v, sem_a, sem_b):
        pltpu.make_async_copy(a_hbm.at[pl.ds(0, CHUNK)], a_v, sem_a).wait()
        pltpu.make_async_copy(bc_hbm.at[pl.ds(0, CHUNK)], bc_v, sem_b).wait()

    def _compute(row_v, a_v, bc_v):
        @plsc.parallel_loop(0, CHUNK, 1, unroll=4)
        def _edge(e):
            parts = []
            for k in range(DIM // 16):
                av = a_v[e, pl.ds(k * 16, 16)]
                bv = bc_v[e, pl.ds(k * 16, 16)]
                parts.append(jnp.maximum(av + bv, 0.0) * w2r[k])
            t0 = (parts[0] + parts[1]) + (parts[2] + parts[3])
            t1 = (parts[4] + parts[5]) + (parts[6] + parts[7])
            dot = jnp.sum(t0 + t1) + b2s
            zv = jnp.broadcast_to(dot, (16,))
            gv = 1.0 / (1.0 + jnp.exp(-zv))
            for k in range(DIM // 16):
                cv = bc_v[e, pl.ds(DIM + k * 16, 16)]
                m_v[e, pl.ds(k * 16, 16)] = gv * cv

        pltpu.sync_copy(m_v, acc.at[row_v], add=True)

    pltpu.sync_copy(idx_hbm.at[cbase], idx_v0)
    _fetch_idx(cbase + 1, idx_v1, sem_i1)
    _issue(idx_v0, a_v0, bc_v0, sem_a0, sem_b0)

    def _pair(i, carry):
        _wait_idx(idx_v1, sem_i1)
        _issue(idx_v1, a_v1, bc_v1, sem_a1, sem_b1)
        _drain(a_v0, bc_v0, sem_a0, sem_b0)
        _compute(idx_v0.at[0], a_v0, bc_v0)
        _fetch_idx(jnp.minimum(cbase + 2 * i + 2, cmax), idx_v0, sem_i0)
        _wait_idx(idx_v0, sem_i0)
        _issue(idx_v0, a_v0, bc_v0, sem_a0, sem_b0)
        _drain(a_v1, bc_v1, sem_a1, sem_b1)
        _compute(idx_v1.at[0], a_v1, bc_v1)
        _fetch_idx(jnp.minimum(cbase + 2 * i + 3, cmax), idx_v1, sem_i1)
        return carry

    lax.fori_loop(0, N_PAIRS, _pair, 0)
    _wait_idx(idx_v1, sem_i1)
    _drain(a_v0, bc_v0, sem_a0, sem_b0)

    plsc.subcore_barrier()

    def _cp(j, carry):
        r0 = s * ROWS_PER_SUB + j * 25
        pltpu.sync_copy(acc.at[pl.ds(r0, 25)], m_v.at[pl.ds(0, 25)])
        pltpu.sync_copy(m_v.at[pl.ds(0, 25)], out_hbm.at[c, pl.ds(r0, 25)])
        return carry

    lax.fori_loop(0, ROWS_PER_SUB // 25, _cp, 0)


def _final_body(x_ref, acc_ref, gamma_ref, beta_ref, o_ref):
    sacc = acc_ref[0] + acc_ref[1]
    msg = sacc[:, :DIM]
    deg = jnp.maximum(sacc[:, DIM:DIM + 1], 1.0)
    y = x_ref[...] + msg / deg
    mean = jnp.mean(y, axis=1, keepdims=True)
    cent = y - mean
    var = jnp.mean(cent * cent, axis=1, keepdims=True)
    yn = cent * lax.rsqrt(var + 1e-5)
    o_ref[...] = jnp.maximum(gamma_ref[...] * yn + beta_ref[...], 0.0)


_final = pl.pallas_call(
    _final_body,
    grid=(N // ROW_BLK,),
    in_specs=[
        pl.BlockSpec((ROW_BLK, DIM), lambda i: (i, 0)),
        pl.BlockSpec((NC, ROW_BLK, ACC_W), lambda i: (0, i, 0)),
        pl.BlockSpec((1, DIM), lambda i: (0, 0)),
        pl.BlockSpec((1, DIM), lambda i: (0, 0)),
    ],
    out_specs=pl.BlockSpec((ROW_BLK, DIM), lambda i: (i, 0)),
    out_shape=jax.ShapeDtypeStruct((N, DIM), jnp.float32),
)


def kernel(x, edge_index, W, W1, b1, W2, b2, gamma, beta):
    wh = jnp.concatenate([W1[:DIM], W1[DIM:], W], axis=1)
    bh = jnp.concatenate([jnp.zeros((DIM,), jnp.float32), b1,
                          jnp.zeros((DIM,), jnp.float32)])[None]
    w2b2 = (jnp.zeros((ACC_W,), jnp.float32)
            .at[:DIM].set(W2[:, 0]).at[DIM].set(b2[0]))

    idx_packed = jnp.transpose(
        edge_index.reshape(2, NW, EDGES_PER_W // CHUNK, CHUNK),
        (1, 2, 0, 3)).reshape(NW * (EDGES_PER_W // CHUNK), 2, CHUNK)

    a_arr, bc_arr = _node_mm(x, wh, bh)
    acc = _edge_sc(idx_packed, a_arr, bc_arr, w2b2)
    return _final(x, acc, gamma[None], beta[None])


# speedup vs baseline: 1.3437x; 1.0487x over previous
import functools

import jax
import jax.numpy as jnp
from jax import lax
from jax.experimental import pallas as pl
from jax.experimental.pallas import tpu as pltpu
from jax.experimental.pallas import tpu_sc as plsc

N = 10000
DIM = 128
E = 320000

NC = 2
NS = 16
NW = NC * NS

ACC_W = 144
N_PAD = 10000
ROWS_PER_SUB = N_PAD // NS
CHUNK = 40
EDGES_PER_W = E // NW
N_PAIRS = EDGES_PER_W // (2 * CHUNK)

ROW_BLK = 2000


def _node_mm_body(x_ref, wh_ref, bh_ref, a_ref, bc_ref):
    y = jnp.dot(x_ref[...], wh_ref[...], preferred_element_type=jnp.float32)
    y = y + bh_ref[...]
    a_ref[...] = y[:, :DIM]
    bc_ref[...] = y[:, DIM:]


_node_mm = pl.pallas_call(
    _node_mm_body,
    grid=(N // ROW_BLK,),
    in_specs=[
        pl.BlockSpec((ROW_BLK, DIM), lambda i: (i, 0)),
        pl.BlockSpec((DIM, 3 * DIM), lambda i: (0, 0)),
        pl.BlockSpec((1, 3 * DIM), lambda i: (0, 0)),
    ],
    out_specs=[
        pl.BlockSpec((ROW_BLK, DIM), lambda i: (i, 0)),
        pl.BlockSpec((ROW_BLK, 2 * DIM), lambda i: (i, 0)),
    ],
    out_shape=[
        jax.ShapeDtypeStruct((N, DIM), jnp.float32),
        jax.ShapeDtypeStruct((N, 2 * DIM), jnp.float32),
    ],
)


_sc_mesh = plsc.VectorSubcoreMesh(core_axis_name="c", subcore_axis_name="s")


@functools.partial(
    pl.kernel,
    out_type=jax.ShapeDtypeStruct((NC, N_PAD, ACC_W), jnp.float32),
    mesh=_sc_mesh,
    compiler_params=pltpu.CompilerParams(needs_layout_passes=False,
                                         use_tc_tiling_on_sc=False),
    scratch_types=[
        pltpu.VMEM((2, CHUNK), jnp.int32),
        pltpu.VMEM((2, CHUNK), jnp.int32),
        pltpu.VMEM((2, CHUNK), jnp.int32),
        pltpu.VMEM((2, CHUNK), jnp.int32),
        pltpu.VMEM((CHUNK, DIM), jnp.float32),
        pltpu.VMEM((CHUNK, 2 * DIM), jnp.float32),
        pltpu.VMEM((CHUNK, DIM), jnp.float32),
        pltpu.VMEM((CHUNK, 2 * DIM), jnp.float32),
        pltpu.VMEM((CHUNK, ACC_W), jnp.float32),
        pltpu.VMEM((ACC_W,), jnp.float32),
        pltpu.VMEM_SHARED((N_PAD, ACC_W), jnp.float32),
        pltpu.SemaphoreType.DMA,
        pltpu.SemaphoreType.DMA,
        pltpu.SemaphoreType.DMA,
        pltpu.SemaphoreType.DMA,
        pltpu.SemaphoreType.DMA,
        pltpu.SemaphoreType.DMA,
        pltpu.SemaphoreType.DMA,
        pltpu.SemaphoreType.DMA,
    ],
)
def _edge_sc(idx_hbm, a_hbm, bc_hbm, w2b2_hbm, out_hbm,
             idx_v0, idx_v1, idx_v2, idx_v3, a_v0, bc_v0, a_v1, bc_v1,
             m_v, w2_v, acc, sem_a0, sem_b0, sem_a1, sem_b1,
             sem_i0, sem_i1, sem_i2, sem_i3):
    c = lax.axis_index("c")
    s = lax.axis_index("s")
    wid = s * NC + c

    zeros16 = jnp.zeros((16,), jnp.float32)

    def _zrow(e, carry):
        for k in range(ACC_W // 16):
            m_v[e, pl.ds(k * 16, 16)] = zeros16
        return carry

    lax.fori_loop(0, CHUNK, _zrow, 0)

    def _zacc(j, carry):
        pltpu.sync_copy(m_v.at[pl.ds(0, 25)],
                        acc.at[pl.ds(s * ROWS_PER_SUB + j * 25, 25)])
        return carry

    lax.fori_loop(0, ROWS_PER_SUB // 25, _zacc, 0)

    deg16 = jnp.where(lax.iota(jnp.int32, 16) == 0, 1.0, 0.0).astype(jnp.float32)

    def _deg1(e, carry):
        m_v[e, pl.ds(DIM, 16)] = deg16
        return carry

    lax.fori_loop(0, CHUNK, _deg1, 0)

    pltpu.sync_copy(w2b2_hbm, w2_v)
    w2r = [w2_v[pl.ds(k * 16, 16)] for k in range(DIM // 16)]
    b2s = w2_v[pl.ds(DIM, 16)][0]

    plsc.subcore_barrier()

    cbase = wid * (EDGES_PER_W // CHUNK)
    cmax = cbase + EDGES_PER_W // CHUNK - 1

    def _fetch_idx(cnum, idx_v, sem):
        pltpu.async_copy(idx_hbm.at[cnum], idx_v, sem)

    def _wait_idx(idx_v, sem):
        pltpu.make_async_copy(idx_hbm.at[0], idx_v, sem).wait()

    def _issue(idx_v, a_v, bc_v, sem_a, sem_b):
        pltpu.async_copy(a_hbm.at[idx_v.at[0]], a_v, sem_a)
        pltpu.async_copy(bc_hbm.at[idx_v.at[1]], bc_v, sem_b)

    def _drain(a_v, bc_v, sem_a, sem_b):
        pltpu.make_async_copy(a_hbm.at[pl.ds(0, CHUNK)], a_v, sem_a).wait()
        pltpu.make_async_copy(bc_hbm.at[pl.ds(0, CHUNK)], bc_v, sem_b).wait()

    def _compute(row_v, a_v, bc_v):
        @plsc.parallel_loop(0, CHUNK, 1, unroll=4)
        def _edge(e):
            parts = []
            for k in range(DIM // 16):
                av = a_v[e, pl.ds(k * 16, 16)]
                bv = bc_v[e, pl.ds(k * 16, 16)]
                parts.append(jnp.maximum(av + bv, 0.0) * w2r[k])
            t0 = (parts[0] + parts[1]) + (parts[2] + parts[3])
            t1 = (parts[4] + parts[5]) + (parts[6] + parts[7])
            dot = jnp.sum(t0 + t1) + b2s
            zv = jnp.broadcast_to(dot, (16,))
            gv = 1.0 / (1.0 + jnp.exp(-zv))
            for k in range(DIM // 16):
                cv = bc_v[e, pl.ds(DIM + k * 16, 16)]
                m_v[e, pl.ds(k * 16, 16)] = gv * cv

        pltpu.sync_copy(m_v, acc.at[row_v], add=True)

    idxs = (idx_v0, idx_v1, idx_v2, idx_v3)
    isems = (sem_i0, sem_i1, sem_i2, sem_i3)
    gbufs = ((a_v0, bc_v0, sem_a0, sem_b0), (a_v1, bc_v1, sem_a1, sem_b1))

    pltpu.sync_copy(idx_hbm.at[cbase], idx_v0)
    for k in range(1, 4):
        _fetch_idx(cbase + k, idxs[k], isems[k])
    _issue(idx_v0, a_v0, bc_v0, sem_a0, sem_b0)

    def _quad(i, carry):
        for k in range(4):
            idx_n, sem_n = idxs[(k + 1) % 4], isems[(k + 1) % 4]
            a_c, bc_c, sa_c, sb_c = gbufs[k % 2]
            a_n, bc_n, sa_n, sb_n = gbufs[(k + 1) % 2]
            _wait_idx(idx_n, sem_n)
            _issue(idx_n, a_n, bc_n, sa_n, sb_n)
            _drain(a_c, bc_c, sa_c, sb_c)
            _compute(idxs[k].at[0], a_c, bc_c)
            _fetch_idx(jnp.minimum(cbase + 4 * i + 4 + k, cmax),
                       idxs[k], isems[k])
        return carry

    lax.fori_loop(0, (EDGES_PER_W // CHUNK - 2) // 4, _quad, 0)

    _wait_idx(idx_v1, sem_i1)
    _issue(idx_v1, a_v1, bc_v1, sem_a1, sem_b1)
    _drain(a_v0, bc_v0, sem_a0, sem_b0)
    _compute(idx_v0.at[0], a_v0, bc_v0)
    _drain(a_v1, bc_v1, sem_a1, sem_b1)
    _compute(idx_v1.at[0], a_v1, bc_v1)
    _wait_idx(idx_v2, sem_i2)
    _wait_idx(idx_v3, sem_i3)

    plsc.subcore_barrier()

    def _cp(j, carry):
        r0 = s * ROWS_PER_SUB + j * 25
        pltpu.sync_copy(acc.at[pl.ds(r0, 25)], m_v.at[pl.ds(0, 25)])
        pltpu.sync_copy(m_v.at[pl.ds(0, 25)], out_hbm.at[c, pl.ds(r0, 25)])
        return carry

    lax.fori_loop(0, ROWS_PER_SUB // 25, _cp, 0)


def _final_body(x_ref, acc_ref, gamma_ref, beta_ref, o_ref):
    sacc = acc_ref[0] + acc_ref[1]
    msg = sacc[:, :DIM]
    deg = jnp.maximum(sacc[:, DIM:DIM + 1], 1.0)
    y = x_ref[...] + msg / deg
    mean = jnp.mean(y, axis=1, keepdims=True)
    cent = y - mean
    var = jnp.mean(cent * cent, axis=1, keepdims=True)
    yn = cent * lax.rsqrt(var + 1e-5)
    o_ref[...] = jnp.maximum(gamma_ref[...] * yn + beta_ref[...], 0.0)


_final = pl.pallas_call(
    _final_body,
    grid=(N // ROW_BLK,),
    in_specs=[
        pl.BlockSpec((ROW_BLK, DIM), lambda i: (i, 0)),
        pl.BlockSpec((NC, ROW_BLK, ACC_W), lambda i: (0, i, 0)),
        pl.BlockSpec((1, DIM), lambda i: (0, 0)),
        pl.BlockSpec((1, DIM), lambda i: (0, 0)),
    ],
    out_specs=pl.BlockSpec((ROW_BLK, DIM), lambda i: (i, 0)),
    out_shape=jax.ShapeDtypeStruct((N, DIM), jnp.float32),
)


def kernel(x, edge_index, W, W1, b1, W2, b2, gamma, beta):
    wh = jnp.concatenate([W1[:DIM], W1[DIM:], W], axis=1)
    bh = jnp.concatenate([jnp.zeros((DIM,), jnp.float32), b1,
                          jnp.zeros((DIM,), jnp.float32)])[None]
    w2b2 = (jnp.zeros((ACC_W,), jnp.float32)
            .at[:DIM].set(W2[:, 0]).at[DIM].set(b2[0]))

    idx_packed = jnp.transpose(
        edge_index.reshape(2, NW, EDGES_PER_W // CHUNK, CHUNK),
        (1, 2, 0, 3)).reshape(NW * (EDGES_PER_W // CHUNK), 2, CHUNK)

    a_arr, bc_arr = _node_mm(x, wh, bh)
    acc = _edge_sc(idx_packed, a_arr, bc_arr, w2b2)
    return _final(x, acc, gamma[None], beta[None])


# speedup vs baseline: 1.4108x; 1.0500x over previous
import functools

import jax
import jax.numpy as jnp
from jax import lax
from jax.experimental import pallas as pl
from jax.experimental.pallas import tpu as pltpu
from jax.experimental.pallas import tpu_sc as plsc

N = 10000
DIM = 128
E = 320000

NC = 2
NS = 16
NW = NC * NS

ACC_W = 144
N_PAD = 10000
ROWS_PER_SUB = N_PAD // NS
CHUNK = 40
EDGES_PER_W = E // NW
N_PAIRS = EDGES_PER_W // (2 * CHUNK)

ROW_BLK = 2000


def _node_mm_body(x_ref, wh_ref, bh_ref, a_ref, bc_ref):
    y = jnp.dot(x_ref[...], wh_ref[...], preferred_element_type=jnp.float32)
    y = y + bh_ref[...]
    a_ref[...] = y[:, :DIM]
    bc_ref[...] = y[:, DIM:]


_node_mm = pl.pallas_call(
    _node_mm_body,
    grid=(N // ROW_BLK,),
    in_specs=[
        pl.BlockSpec((ROW_BLK, DIM), lambda i: (i, 0)),
        pl.BlockSpec((DIM, 3 * DIM), lambda i: (0, 0)),
        pl.BlockSpec((1, 3 * DIM), lambda i: (0, 0)),
    ],
    out_specs=[
        pl.BlockSpec((ROW_BLK, DIM), lambda i: (i, 0)),
        pl.BlockSpec((ROW_BLK, 2 * DIM), lambda i: (i, 0)),
    ],
    out_shape=[
        jax.ShapeDtypeStruct((N, DIM), jnp.float32),
        jax.ShapeDtypeStruct((N, 2 * DIM), jnp.float32),
    ],
)


_sc_mesh = plsc.VectorSubcoreMesh(core_axis_name="c", subcore_axis_name="s")


@functools.partial(
    pl.kernel,
    out_type=jax.ShapeDtypeStruct((NC, N_PAD, ACC_W), jnp.float32),
    mesh=_sc_mesh,
    compiler_params=pltpu.CompilerParams(needs_layout_passes=False,
                                         use_tc_tiling_on_sc=False),
    scratch_types=[
        pltpu.VMEM((2, CHUNK), jnp.int32),
        pltpu.VMEM((2, CHUNK), jnp.int32),
        pltpu.VMEM((2, CHUNK), jnp.int32),
        pltpu.VMEM((2, CHUNK), jnp.int32),
        pltpu.VMEM((CHUNK, DIM), jnp.float32),
        pltpu.VMEM((CHUNK, 2 * DIM), jnp.float32),
        pltpu.VMEM((CHUNK, DIM), jnp.float32),
        pltpu.VMEM((CHUNK, 2 * DIM), jnp.float32),
        pltpu.VMEM((CHUNK, ACC_W), jnp.float32),
        pltpu.VMEM((ACC_W,), jnp.float32),
        pltpu.VMEM_SHARED((N_PAD, ACC_W), jnp.float32),
        pltpu.SemaphoreType.DMA,
        pltpu.SemaphoreType.DMA,
        pltpu.SemaphoreType.DMA,
        pltpu.SemaphoreType.DMA,
        pltpu.SemaphoreType.DMA,
        pltpu.SemaphoreType.DMA,
        pltpu.SemaphoreType.DMA,
        pltpu.SemaphoreType.DMA,
        pltpu.SemaphoreType.DMA,
    ],
)
def _edge_sc(idx_hbm, a_hbm, bc_hbm, w2b2_hbm, out_hbm,
             idx_v0, idx_v1, idx_v2, idx_v3, a_v0, bc_v0, a_v1, bc_v1,
             m_v, w2_v, acc, sem_a0, sem_b0, sem_a1, sem_b1,
             sem_i0, sem_i1, sem_i2, sem_i3, sem_s):
    c = lax.axis_index("c")
    s = lax.axis_index("s")
    wid = s * NC + c

    zeros16 = jnp.zeros((16,), jnp.float32)

    def _zrow(e, carry):
        for k in range(ACC_W // 16):
            m_v[e, pl.ds(k * 16, 16)] = zeros16
        return carry

    lax.fori_loop(0, CHUNK, _zrow, 0)

    def _zacc(j, carry):
        pltpu.sync_copy(m_v.at[pl.ds(0, 25)],
                        acc.at[pl.ds(s * ROWS_PER_SUB + j * 25, 25)])
        return carry

    lax.fori_loop(0, ROWS_PER_SUB // 25, _zacc, 0)

    deg16 = jnp.where(lax.iota(jnp.int32, 16) == 0, 1.0, 0.0).astype(jnp.float32)

    def _deg1(e, carry):
        m_v[e, pl.ds(DIM, 16)] = deg16
        return carry

    lax.fori_loop(0, CHUNK, _deg1, 0)

    pltpu.sync_copy(w2b2_hbm, w2_v)
    w2r = [w2_v[pl.ds(k * 16, 16)] for k in range(DIM // 16)]
    b2s = w2_v[pl.ds(DIM, 16)][0]

    plsc.subcore_barrier()

    cbase = wid * (EDGES_PER_W // CHUNK)
    cmax = cbase + EDGES_PER_W // CHUNK - 1

    def _fetch_idx(cnum, idx_v, sem):
        pltpu.async_copy(idx_hbm.at[cnum], idx_v, sem)

    def _wait_idx(idx_v, sem):
        pltpu.make_async_copy(idx_hbm.at[0], idx_v, sem).wait()

    def _issue(idx_v, a_v, bc_v, sem_a, sem_b):
        pltpu.async_copy(a_hbm.at[idx_v.at[0]], a_v, sem_a)
        pltpu.async_copy(bc_hbm.at[idx_v.at[1]], bc_v, sem_b)

    def _drain(a_v, bc_v, sem_a, sem_b):
        pltpu.make_async_copy(a_hbm.at[pl.ds(0, CHUNK)], a_v, sem_a).wait()
        pltpu.make_async_copy(bc_hbm.at[pl.ds(0, CHUNK)], bc_v, sem_b).wait()

    def _compute(row_v, a_v, bc_v):
        @plsc.parallel_loop(0, CHUNK, 1, unroll=4)
        def _edge(e):
            parts = []
            for k in range(DIM // 16):
                av = a_v[e, pl.ds(k * 16, 16)]
                bv = bc_v[e, pl.ds(k * 16, 16)]
                parts.append(jnp.maximum(av + bv, 0.0) * w2r[k])
            t0 = (parts[0] + parts[1]) + (parts[2] + parts[3])
            t1 = (parts[4] + parts[5]) + (parts[6] + parts[7])
            dot = jnp.sum(t0 + t1) + b2s
            zv = jnp.broadcast_to(dot, (16,))
            gv = 1.0 / (1.0 + jnp.exp(-zv))
            for k in range(DIM // 16):
                cv = bc_v[e, pl.ds(DIM + k * 16, 16)]
                m_v[e, pl.ds(k * 16, 16)] = gv * cv

        pltpu.async_copy(m_v, acc.at[row_v], sem_s, add=True)

    def _drain_scatter():
        pltpu.make_async_copy(m_v, acc.at[idx_v0.at[0]], sem_s).wait()

    idxs = (idx_v0, idx_v1, idx_v2, idx_v3)
    isems = (sem_i0, sem_i1, sem_i2, sem_i3)
    gbufs = ((a_v0, bc_v0, sem_a0, sem_b0), (a_v1, bc_v1, sem_a1, sem_b1))

    pltpu.sync_copy(idx_hbm.at[cbase], idx_v0)
    for k in range(1, 3):
        _fetch_idx(cbase + k, idxs[k], isems[k])
    _issue(idx_v0, a_v0, bc_v0, sem_a0, sem_b0)

    def _quad(i, carry):
        for k in range(4):
            idx_n, sem_n = idxs[(k + 1) % 4], isems[(k + 1) % 4]
            a_c, bc_c, sa_c, sb_c = gbufs[k % 2]
            a_n, bc_n, sa_n, sb_n = gbufs[(k + 1) % 2]
            _wait_idx(idx_n, sem_n)
            _issue(idx_n, a_n, bc_n, sa_n, sb_n)
            _drain(a_c, bc_c, sa_c, sb_c)
            if k == 0:
                @pl.when(i > 0)
                def _():
                    _drain_scatter()
            else:
                _drain_scatter()
            _compute(idxs[k].at[0], a_c, bc_c)
            _fetch_idx(jnp.minimum(cbase + 4 * i + k + 3, cmax),
                       idxs[(k + 3) % 4], isems[(k + 3) % 4])
        return carry

    lax.fori_loop(0, (EDGES_PER_W // CHUNK - 2) // 4, _quad, 0)

    _wait_idx(idx_v1, sem_i1)
    _issue(idx_v1, a_v1, bc_v1, sem_a1, sem_b1)
    _drain(a_v0, bc_v0, sem_a0, sem_b0)
    _drain_scatter()
    _compute(idx_v0.at[0], a_v0, bc_v0)
    _drain(a_v1, bc_v1, sem_a1, sem_b1)
    _drain_scatter()
    _compute(idx_v1.at[0], a_v1, bc_v1)
    _drain_scatter()
    _wait_idx(idx_v2, sem_i2)

    plsc.subcore_barrier()

    def _cp(j, carry):
        r0 = s * ROWS_PER_SUB + j * 25
        pltpu.sync_copy(acc.at[pl.ds(r0, 25)], m_v.at[pl.ds(0, 25)])
        pltpu.sync_copy(m_v.at[pl.ds(0, 25)], out_hbm.at[c, pl.ds(r0, 25)])
        return carry

    lax.fori_loop(0, ROWS_PER_SUB // 25, _cp, 0)


def _final_body(x_ref, acc_ref, gamma_ref, beta_ref, o_ref):
    sacc = acc_ref[0] + acc_ref[1]
    msg = sacc[:, :DIM]
    deg = jnp.maximum(sacc[:, DIM:DIM + 1], 1.0)
    y = x_ref[...] + msg / deg
    mean = jnp.mean(y, axis=1, keepdims=True)
    cent = y - mean
    var = jnp.mean(cent * cent, axis=1, keepdims=True)
    yn = cent * lax.rsqrt(var + 1e-5)
    o_ref[...] = jnp.maximum(gamma_ref[...] * yn + beta_ref[...], 0.0)


_final = pl.pallas_call(
    _final_body,
    grid=(N // ROW_BLK,),
    in_specs=[
        pl.BlockSpec((ROW_BLK, DIM), lambda i: (i, 0)),
        pl.BlockSpec((NC, ROW_BLK, ACC_W), lambda i: (0, i, 0)),
        pl.BlockSpec((1, DIM), lambda i: (0, 0)),
        pl.BlockSpec((1, DIM), lambda i: (0, 0)),
    ],
    out_specs=pl.BlockSpec((ROW_BLK, DIM), lambda i: (i, 0)),
    out_shape=jax.ShapeDtypeStruct((N, DIM), jnp.float32),
)


def kernel(x, edge_index, W, W1, b1, W2, b2, gamma, beta):
    wh = jnp.concatenate([W1[:DIM], W1[DIM:], W], axis=1)
    bh = jnp.concatenate([jnp.zeros((DIM,), jnp.float32), b1,
                          jnp.zeros((DIM,), jnp.float32)])[None]
    w2b2 = (jnp.zeros((ACC_W,), jnp.float32)
            .at[:DIM].set(W2[:, 0]).at[DIM].set(b2[0]))

    idx_packed = jnp.transpose(
        edge_index.reshape(2, NW, EDGES_PER_W // CHUNK, CHUNK),
        (1, 2, 0, 3)).reshape(NW * (EDGES_PER_W // CHUNK), 2, CHUNK)

    a_arr, bc_arr = _node_mm(x, wh, bh)
    acc = _edge_sc(idx_packed, a_arr, bc_arr, w2b2)
    return _final(x, acc, gamma[None], beta[None])
